# baseline (speedup 1.0000x reference)
import functools

import jax
import jax.numpy as jnp
from jax import lax
from jax.experimental import pallas as pl
from jax.experimental.pallas import tpu as pltpu
from jax.experimental.pallas import tpu_sc as plsc

DIM = 64
LANES = 16
NUM_CORES = 2
NUM_SUBCORES = 16
NUM_WORKERS = NUM_CORES * NUM_SUBCORES
CHUNK = 128
NBUF = 2


def _sc_scores(uf, pf, nf, upar, ppar, npar, ut2, it2, batch):
    b_per_w = batch // NUM_WORKERS
    n_chunks = b_per_w // CHUNK

    mesh = plsc.VectorSubcoreMesh(core_axis_name="c", subcore_axis_name="s")

    @functools.partial(
        pl.kernel,
        mesh=mesh,
        out_type=jax.ShapeDtypeStruct((NUM_WORKERS, b_per_w), jnp.float32),
        compiler_params=pltpu.CompilerParams(use_tc_tiling_on_sc=True,
                                             needs_layout_passes=False),
        scratch_types=[
            pltpu.VMEM((n_chunks, CHUNK), jnp.int32),
            pltpu.VMEM((n_chunks, CHUNK), jnp.int32),
            pltpu.VMEM((n_chunks, CHUNK), jnp.int32),
            pltpu.VMEM((n_chunks, CHUNK), jnp.int32),
            pltpu.VMEM((n_chunks, CHUNK), jnp.int32),
            pltpu.VMEM((n_chunks, CHUNK), jnp.int32),
            pltpu.VMEM((NBUF, CHUNK, 2 * DIM), jnp.float32),
            pltpu.VMEM((NBUF, CHUNK, 2 * DIM), jnp.float32),
            pltpu.VMEM((NBUF, CHUNK, 2 * DIM), jnp.float32),
            pltpu.VMEM((b_per_w,), jnp.float32),
            pltpu.SemaphoreType.DMA,
            pltpu.SemaphoreType.DMA,
        ],
    )
    def sc_kernel(uf_hbm, pf_hbm, nf_hbm, up_hbm, pp_hbm, np_hbm,
                  ut_hbm, it_hbm, out_hbm,
                  iu, ip, inn, pu, pp, pn, u_v, p_v, n_v, s_v, sem0, sem1):
        wid = lax.axis_index("s") * NUM_CORES + lax.axis_index("c")
        sems = [sem0, sem1]

        pltpu.sync_copy(uf_hbm.at[wid], iu)
        pltpu.sync_copy(pf_hbm.at[wid], ip)
        pltpu.sync_copy(nf_hbm.at[wid], inn)
        pltpu.sync_copy(up_hbm.at[wid], pu)
        pltpu.sync_copy(pp_hbm.at[wid], pp)
        pltpu.sync_copy(np_hbm.at[wid], pn)

        def start(j, slot):
            return [
                pltpu.async_copy(ut_hbm.at[iu.at[j]], u_v.at[slot], sems[slot]),
                pltpu.async_copy(it_hbm.at[ip.at[j]], p_v.at[slot], sems[slot]),
                pltpu.async_copy(it_hbm.at[inn.at[j]], n_v.at[slot], sems[slot]),
            ]

        def compute(j, slot):
            for g in range(CHUNK // LANES):
                sl = pl.ds(g * LANES, LANES)
                rows = g * LANES + lax.iota(jnp.int32, LANES)
                cu = DIM * pu[j, sl]
                cp = DIM * pp[j, sl]
                cn = DIM * pn[j, sl]
                acc = jnp.zeros((LANES,), jnp.float32)
                for d in range(DIM):
                    du = plsc.load_gather(u_v.at[slot], [rows, cu + d])
                    dp = plsc.load_gather(p_v.at[slot], [rows, cp + d])
                    dn = plsc.load_gather(n_v.at[slot], [rows, cn + d])
                    acc = acc + du * (dp - dn)
                s_v[pl.ds(j * CHUNK + g * LANES, LANES)] = acc

        inflight = {0: start(0, 0)}
        for j in range(n_chunks):
            if j + 1 < n_chunks:
                inflight[j + 1] = start(j + 1, (j + 1) % NBUF)
            for c in inflight.pop(j):
                c.wait()
            compute(j, j % NBUF)

        pltpu.sync_copy(s_v, out_hbm.at[wid])

    return sc_kernel(uf, pf, nf, upar, ppar, npar, ut2, it2)


def _tc_loss_body(w_ref, o_ref):
    tmp = w_ref[...]
    bpr = jnp.maximum(-tmp, 0.0) + jnp.log1p(jnp.exp(-jnp.abs(tmp)))
    o_ref[0, 0] = jnp.sum(bpr)


def kernel(user, pos, neg, user_table, item_table):
    batch = user.shape[0]
    b_per_w = batch // NUM_WORKERS
    n_chunks = b_per_w // CHUNK
    shape3 = (NUM_WORKERS, n_chunks, CHUNK)

    user = user.astype(jnp.int32)
    pos = pos.astype(jnp.int32)
    neg = neg.astype(jnp.int32)
    ut2 = user_table.reshape(user_table.shape[0] // 2, 2 * DIM)
    it2 = item_table.reshape(item_table.shape[0] // 2, 2 * DIM)

    scores = _sc_scores(
        (user >> 1).reshape(shape3), (pos >> 1).reshape(shape3),
        (neg >> 1).reshape(shape3), (user & 1).reshape(shape3),
        (pos & 1).reshape(shape3), (neg & 1).reshape(shape3),
        ut2, it2, batch)
    loss = pl.pallas_call(
        _tc_loss_body,
        out_shape=jax.ShapeDtypeStruct((1, 1), jnp.float32),
        in_specs=[pl.BlockSpec(memory_space=pltpu.VMEM)],
        out_specs=pl.BlockSpec(memory_space=pltpu.SMEM),
    )(scores)
    return loss[0, 0]
